# baseline (device time: 17311 ns/iter reference)
import jax
import jax.numpy as jnp
from jax import lax
from jax.experimental import pallas as pl
from jax.experimental.pallas import tpu as pltpu

ROWS = 1024
COLS = 512
CHUNK = 128
MAX_CHUNKS = ROWS // CHUNK


def kernel(x, dest):
    p = lax.axis_index("y")

    k = jnp.sum((dest == p).astype(jnp.int32))
    s = ROWS - k
    n_chunks = (s + CHUNK - 1) // CHUNK
    base_local = p * s
    send_w0 = (1 - p) * k
    shift = jnp.where(p == 0, k, s)
    meta = jnp.stack([n_chunks, base_local, k, shift, send_w0]).astype(
        jnp.int32
    )

    def body(meta_ref, dest_ref, x_ref, out_ref, xbf_ref, staged_ref,
             recv_ref, send_sems, recv_sems):
        my_x = lax.axis_index("x")
        my_y = lax.axis_index("y")
        partner = (my_x, 1 - my_y)

        barrier = pltpu.get_barrier_semaphore()
        pl.semaphore_signal(
            barrier, inc=1, device_id=partner,
            device_id_type=pl.DeviceIdType.MESH,
        )

        xbf_ref[:, :] = x_ref[:, :].astype(jnp.bfloat16)

        lane = lax.broadcasted_iota(jnp.int32, (1, ROWS), 1)
        is_send = (dest_ref[:, :] != my_y).astype(jnp.float32)
        cs = is_send
        d = 1
        while d < ROWS:
            cs = cs + jnp.where(lane >= d, pltpu.roll(cs, d, 1), 0.0)
            d *= 2
        send_rank = cs - is_send
        keep_rank = lane.astype(jnp.float32) - send_rank

        base_l = meta_ref[1]
        w0 = meta_ref[4]
        pos = jnp.where(
            is_send > 0.0,
            w0.astype(jnp.float32) + send_rank,
            base_l.astype(jnp.float32) + keep_rank,
        )
        pos_i = pos.astype(jnp.int32)

        n = meta_ref[0]

        def chunk_off(j):
            off = jnp.where(my_y == 0, ROWS - (j + 1) * CHUNK, j * CHUNK)
            return pl.multiple_of(off, CHUNK)

        for j in range(MAX_CHUNKS):
            off = chunk_off(j)
            tgt = off + lax.broadcasted_iota(jnp.int32, (CHUNK, ROWS), 0)
            onehot = (tgt == pos_i[0, :][None, :]).astype(jnp.bfloat16)
            block = jnp.dot(
                onehot, xbf_ref[:, :], preferred_element_type=jnp.float32
            )
            staged_ref[pl.ds(off, CHUNK), :] = block.astype(jnp.bfloat16)
            if j == 0:
                pl.semaphore_wait(barrier, 1)

            @pl.when(j < n)
            def _():
                rdma = pltpu.make_async_remote_copy(
                    src_ref=staged_ref.at[pl.ds(off, CHUNK)],
                    dst_ref=recv_ref.at[pl.ds(off, CHUNK)],
                    send_sem=send_sems.at[j],
                    recv_sem=recv_sems.at[j],
                    device_id=partner,
                    device_id_type=pl.DeviceIdType.MESH,
                )
                rdma.start()

        for j in range(MAX_CHUNKS):
            @pl.when(j < n)
            def _():
                off = chunk_off(j)
                done = pltpu.make_async_remote_copy(
                    src_ref=staged_ref.at[pl.ds(off, CHUNK)],
                    dst_ref=recv_ref.at[pl.ds(off, CHUNK)],
                    send_sem=send_sems.at[j],
                    recv_sem=recv_sems.at[j],
                    device_id=partner,
                    device_id_type=pl.DeviceIdType.MESH,
                )
                done.wait_recv()
                done.wait_send()

        k_ = meta_ref[2]
        shift_ = meta_ref[3]
        rolled = pltpu.roll(recv_ref[:, :], shift_, 0)
        row = lax.broadcasted_iota(jnp.int32, (ROWS, COLS), 0)
        in_local = (row >= base_l) & (row < base_l + k_)
        out_ref[:, :] = jnp.where(
            in_local, staged_ref[:, :], rolled
        ).astype(jnp.float32)

    return pl.pallas_call(
        body,
        out_shape=jax.ShapeDtypeStruct((ROWS, COLS), jnp.float32),
        in_specs=[
            pl.BlockSpec(memory_space=pltpu.SMEM),
            pl.BlockSpec(memory_space=pltpu.VMEM),
            pl.BlockSpec(memory_space=pltpu.VMEM),
        ],
        out_specs=pl.BlockSpec(memory_space=pltpu.VMEM),
        scratch_shapes=[
            pltpu.VMEM((ROWS, COLS), jnp.bfloat16),
            pltpu.VMEM((ROWS, COLS), jnp.bfloat16),
            pltpu.VMEM((ROWS, COLS), jnp.bfloat16),
            pltpu.SemaphoreType.DMA((MAX_CHUNKS,)),
            pltpu.SemaphoreType.DMA((MAX_CHUNKS,)),
        ],
        compiler_params=pltpu.CompilerParams(collective_id=0),
    )(meta, dest.reshape(1, ROWS), x)


# device time: 16786 ns/iter; 1.0313x vs baseline; 1.0313x over previous
import jax
import jax.numpy as jnp
from jax import lax
from jax.experimental import pallas as pl
from jax.experimental.pallas import tpu as pltpu

ROWS = 1024
COLS = 512
CHUNK = 64
MAX_CHUNKS = ROWS // CHUNK


def kernel(x, dest):
    p = lax.axis_index("y")

    is_keep = (dest == p).astype(jnp.int32)
    is_send = 1 - is_keep
    k = jnp.sum(is_keep)
    s = ROWS - k
    send_rank = jnp.cumsum(is_send) - is_send
    keep_rank = jnp.arange(ROWS, dtype=jnp.int32) - send_rank
    base_local = p * s
    send_w0 = (1 - p) * k
    shift = jnp.where(p == 0, k, s)

    pos = jnp.where(is_keep == 1, base_local + keep_rank, send_w0 + send_rank)
    n_chunks = (s + CHUNK - 1) // CHUNK
    meta = jnp.stack([n_chunks, base_local, k, shift]).astype(jnp.int32)
    x_bf = x.astype(jnp.bfloat16)

    def body(meta_ref, pos_ref, x_ref, out_ref, staged_ref, recv_ref,
             send_sems, recv_sems):
        my_x = lax.axis_index("x")
        my_y = lax.axis_index("y")
        partner = (my_x, 1 - my_y)

        barrier = pltpu.get_barrier_semaphore()
        pl.semaphore_signal(
            barrier, inc=1, device_id=partner,
            device_id_type=pl.DeviceIdType.MESH,
        )

        n = meta_ref[0]

        def chunk_off(j):
            off = jnp.where(my_y == 0, ROWS - (j + 1) * CHUNK, j * CHUNK)
            return pl.multiple_of(off, CHUNK)

        for j in range(MAX_CHUNKS):
            off = chunk_off(j)
            tgt = off + lax.broadcasted_iota(jnp.int32, (CHUNK, ROWS), 0)
            onehot = (tgt == pos_ref[0, :][None, :]).astype(jnp.bfloat16)
            block = jnp.dot(
                onehot, x_ref[:, :], preferred_element_type=jnp.float32
            )
            staged_ref[pl.ds(off, CHUNK), :] = block.astype(jnp.bfloat16)
            if j == 0:
                pl.semaphore_wait(barrier, 1)

            @pl.when(j < n)
            def _():
                rdma = pltpu.make_async_remote_copy(
                    src_ref=staged_ref.at[pl.ds(off, CHUNK)],
                    dst_ref=recv_ref.at[pl.ds(off, CHUNK)],
                    send_sem=send_sems.at[j],
                    recv_sem=recv_sems.at[j],
                    device_id=partner,
                    device_id_type=pl.DeviceIdType.MESH,
                )
                rdma.start()

        for j in range(MAX_CHUNKS):
            @pl.when(j < n)
            def _():
                off = chunk_off(j)
                done = pltpu.make_async_remote_copy(
                    src_ref=staged_ref.at[pl.ds(off, CHUNK)],
                    dst_ref=recv_ref.at[pl.ds(off, CHUNK)],
                    send_sem=send_sems.at[j],
                    recv_sem=recv_sems.at[j],
                    device_id=partner,
                    device_id_type=pl.DeviceIdType.MESH,
                )
                done.wait_recv()
                done.wait_send()

        base_l = meta_ref[1]
        k_ = meta_ref[2]
        shift_ = meta_ref[3]
        rolled = pltpu.roll(recv_ref[:, :], shift_, 0)
        row = lax.broadcasted_iota(jnp.int32, (ROWS, COLS), 0)
        in_local = (row >= base_l) & (row < base_l + k_)
        out_ref[:, :] = jnp.where(
            in_local, staged_ref[:, :], rolled
        ).astype(jnp.float32)

    return pl.pallas_call(
        body,
        out_shape=jax.ShapeDtypeStruct((ROWS, COLS), jnp.float32),
        in_specs=[
            pl.BlockSpec(memory_space=pltpu.SMEM),
            pl.BlockSpec(memory_space=pltpu.VMEM),
            pl.BlockSpec(memory_space=pltpu.VMEM),
        ],
        out_specs=pl.BlockSpec(memory_space=pltpu.VMEM),
        scratch_shapes=[
            pltpu.VMEM((ROWS, COLS), jnp.bfloat16),
            pltpu.VMEM((ROWS, COLS), jnp.bfloat16),
            pltpu.SemaphoreType.DMA((MAX_CHUNKS,)),
            pltpu.SemaphoreType.DMA((MAX_CHUNKS,)),
        ],
        compiler_params=pltpu.CompilerParams(collective_id=0),
    )(meta, pos.reshape(1, ROWS), x_bf)


# device time: 16638 ns/iter; 1.0404x vs baseline; 1.0089x over previous
import jax
import jax.numpy as jnp
from jax import lax
from jax.experimental import pallas as pl
from jax.experimental.pallas import tpu as pltpu

ROWS = 1024
COLS = 512
CHUNK = 128
MAX_CHUNKS = ROWS // CHUNK


def kernel(x, dest):
    p = lax.axis_index("y")

    is_keep = (dest == p).astype(jnp.int32)
    is_send = 1 - is_keep
    k = jnp.sum(is_keep)
    s = ROWS - k
    send_rank = jnp.cumsum(is_send) - is_send
    keep_rank = jnp.arange(ROWS, dtype=jnp.int32) - send_rank
    base_local = p * s
    send_w0 = (1 - p) * k
    shift = jnp.where(p == 0, k, s)

    pos = jnp.where(is_keep == 1, base_local + keep_rank, send_w0 + send_rank)
    n_chunks = (s + CHUNK - 1) // CHUNK
    meta = jnp.stack([n_chunks, base_local, k, shift]).astype(jnp.int32)
    x_bf = x.astype(jnp.bfloat16)

    def body(meta_ref, pos_ref, x_ref, out_ref, staged_ref, recv_ref,
             send_sems, recv_sems):
        my_x = lax.axis_index("x")
        my_y = lax.axis_index("y")
        partner = (my_x, 1 - my_y)

        barrier = pltpu.get_barrier_semaphore()
        pl.semaphore_signal(
            barrier, inc=1, device_id=partner,
            device_id_type=pl.DeviceIdType.MESH,
        )

        n = meta_ref[0]

        def chunk_off(j):
            off = jnp.where(my_y == 0, ROWS - (j + 1) * CHUNK, j * CHUNK)
            return pl.multiple_of(off, CHUNK)

        for j in range(MAX_CHUNKS):
            off = chunk_off(j)
            tgt = off + lax.broadcasted_iota(jnp.int32, (CHUNK, ROWS), 0)
            onehot = (tgt == pos_ref[0, :][None, :]).astype(jnp.bfloat16)
            block = jnp.dot(
                onehot, x_ref[:, :], preferred_element_type=jnp.float32
            )
            staged_ref[pl.ds(off, CHUNK), :] = block.astype(jnp.bfloat16)
            if j == 0:
                pl.semaphore_wait(barrier, 1)

            @pl.when(j < n)
            def _():
                rdma = pltpu.make_async_remote_copy(
                    src_ref=staged_ref.at[pl.ds(off, CHUNK)],
                    dst_ref=recv_ref.at[pl.ds(off, CHUNK)],
                    send_sem=send_sems.at[j],
                    recv_sem=recv_sems.at[j],
                    device_id=partner,
                    device_id_type=pl.DeviceIdType.MESH,
                )
                rdma.start()

        for j in range(MAX_CHUNKS):
            @pl.when(j < n)
            def _():
                off = chunk_off(j)
                done = pltpu.make_async_remote_copy(
                    src_ref=staged_ref.at[pl.ds(off, CHUNK)],
                    dst_ref=recv_ref.at[pl.ds(off, CHUNK)],
                    send_sem=send_sems.at[j],
                    recv_sem=recv_sems.at[j],
                    device_id=partner,
                    device_id_type=pl.DeviceIdType.MESH,
                )
                done.wait_recv()
                done.wait_send()

        base_l = meta_ref[1]
        k_ = meta_ref[2]
        shift_ = meta_ref[3]
        rolled = pltpu.roll(recv_ref[:, :], shift_, 0)
        row = lax.broadcasted_iota(jnp.int32, (ROWS, COLS), 0)
        in_local = (row >= base_l) & (row < base_l + k_)
        out_ref[:, :] = jnp.where(
            in_local, staged_ref[:, :], rolled
        ).astype(jnp.float32)

    return pl.pallas_call(
        body,
        out_shape=jax.ShapeDtypeStruct((ROWS, COLS), jnp.float32),
        in_specs=[
            pl.BlockSpec(memory_space=pltpu.SMEM),
            pl.BlockSpec(memory_space=pltpu.VMEM),
            pl.BlockSpec(memory_space=pltpu.VMEM),
        ],
        out_specs=pl.BlockSpec(memory_space=pltpu.VMEM),
        scratch_shapes=[
            pltpu.VMEM((ROWS, COLS), jnp.bfloat16),
            pltpu.VMEM((ROWS, COLS), jnp.bfloat16),
            pltpu.SemaphoreType.DMA((MAX_CHUNKS,)),
            pltpu.SemaphoreType.DMA((MAX_CHUNKS,)),
        ],
        compiler_params=pltpu.CompilerParams(collective_id=0),
    )(meta, pos.reshape(1, ROWS), x_bf)
